# baseline (speedup 1.0000x reference)
import functools

import jax
import jax.numpy as jnp
from jax import lax
from jax.experimental import pallas as pl
from jax.experimental.pallas import tpu as pltpu
from jax.experimental.pallas import tpu_sc as plsc

_N_PRED = 512
_N_CLS = 128
_B = 16
_S = 4096
_NT = _B * _N_CLS
_NW = 32
_PPW = _N_PRED // _NW
_L = 16
_KW = _PPW * _NT
_HW = _N_PRED * _NT
_ZW = 4096
_NCH = _S // _N_CLS


def _sc_hist_iou(pred, targ):
  mesh = plsc.VectorSubcoreMesh(core_axis_name="c", subcore_axis_name="s")

  @functools.partial(
      pl.kernel,
      out_type=jax.ShapeDtypeStruct((_N_PRED, _N_CLS), jnp.float32),
      mesh=mesh,
      compiler_params=pltpu.CompilerParams(
          use_tc_tiling_on_sc=False, needs_layout_passes=False),
      scratch_types=[
          pltpu.VMEM((_S,), jnp.int32),
          pltpu.VMEM((_S,), jnp.int32),
          pltpu.VMEM((_NCH, _N_CLS), jnp.int32),
          pltpu.VMEM((_NT,), jnp.float32),
          pltpu.VMEM((_NT,), jnp.float32),
          pltpu.VMEM((_NT,), jnp.float32),
          pltpu.VMEM((_N_CLS,), jnp.float32),
          pltpu.VMEM((_PPW, _N_CLS), jnp.float32),
          pltpu.VMEM((_ZW,), jnp.float32),
          pltpu.VMEM((_N_CLS,), jnp.float32),
          pltpu.VMEM_SHARED((_HW,), jnp.float32),
          pltpu.VMEM_SHARED((_NT,), jnp.float32),
          pltpu.SemaphoreType.DMA,
          pltpu.SemaphoreType.DMA,
          pltpu.SemaphoreType.DMA,
      ],
  )
  def run(pred_hbm, targ_hbm, out_hbm, pbuf, tbuf, kbuf, hrow0, hrow1, ntv,
          ntloc, outv, zbuf, oneb, hsh, ntsh, semk, semz, sems):
    cid = lax.axis_index("c")
    sid = lax.axis_index("s")
    wid = sid * 2 + cid
    lo = wid * _PPW
    zer = jnp.zeros((_L,), jnp.float32)
    one = jnp.ones((_L,), jnp.float32)

    cpp = pltpu.async_copy(pred_hbm.at[sid], pbuf, semk)
    cpt = pltpu.async_copy(targ_hbm.at[sid], tbuf, semk)

    def zero_z(i, c):
      for k in range(8):
        zbuf[pl.ds(i * (8 * _L) + k * _L, _L)] = zer
      return c
    lax.fori_loop(0, _ZW // (8 * _L), zero_z, 0)
    for k in range(_N_CLS // _L):
      ntloc[pl.ds(k * _L, _L)] = zer
      oneb[pl.ds(k * _L, _L)] = one

    zcopies = [
        pltpu.async_copy(
            zbuf, hsh.at[pl.ds(sid * (_HW // 16) + i * _ZW, _ZW)], semz)
        for i in range(_HW // 16 // _ZW)
    ]
    cpp.wait()
    cpt.wait()

    boff = sid * _N_CLS

    def key_build(j, c):
      for k in range(_N_CLS // _L):
        o = j * _N_CLS + k * _L
        pv = pbuf[pl.ds(o, _L)]
        tv = tbuf[pl.ds(o, _L)]
        kbuf[j, pl.ds(k * _L, _L)] = (pv << 11) + (tv + boff)
        plsc.addupdate_scatter(ntloc, [tv], one)
      return c
    lax.fori_loop(0, _NCH, key_build, 0)
    pltpu.sync_copy(ntloc, ntsh.at[pl.ds(sid * _N_CLS, _N_CLS)])

    for cp in zcopies:
      cp.wait()
    plsc.subcore_barrier()

    scopies = [
        pltpu.async_copy(oneb, hsh.at[kbuf.at[j]], sems, add=True)
        for j in range(_NCH)
    ]
    for cp in scopies:
      cp.wait()
    plsc.subcore_barrier()

    hbase = wid * _KW
    hrows = (hrow0, hrow1)
    cpn = pltpu.async_copy(ntsh, ntv, semz)
    pend = pltpu.async_copy(hsh.at[pl.ds(hbase, _NT)], hrows[0], semk)
    cpn.wait()
    for r in range(_PPW):
      pend.wait()
      if r + 1 < _PPW:
        pend = pltpu.async_copy(
            hsh.at[pl.ds(hbase + (r + 1) * _NT, _NT)],
            hrows[(r + 1) % 2], semk)
      hb = hrows[r % 2]

      def np_sum(v, acc, hb=hb):
        s = acc
        for k in range(8):
          s = s + hb[pl.ds(v * (8 * _L) + k * _L, _L)]
        return s
      np_lanes = lax.fori_loop(0, _NT // (8 * _L), np_sum, zer)
      np_v = jnp.full((_L,), jnp.sum(np_lanes))

      def acc_b(b, accs, hb=hb, np_v=np_v):
        nb = b * _N_CLS
        out = []
        for j in range(_N_CLS // _L):
          h = hb[pl.ds(nb + j * _L, _L)]
          nt = ntv[pl.ds(nb + j * _L, _L)]
          iou = jnp.where(h > 0.0, h / ((np_v + nt) - h), 0.0)
          out.append(accs[j] + iou)
        return tuple(out)

      accs = lax.fori_loop(0, _B, acc_b, tuple(zer for _ in range(_N_CLS // _L)))
      for j in range(_N_CLS // _L):
        outv[r, pl.ds(j * _L, _L)] = accs[j]

    pltpu.sync_copy(outv, out_hbm.at[pl.ds(lo, _PPW)])

  return run(pred, targ)


def _tc_finish(m, targets):
  def body(m_ref, t_ref, o_ref):
    prod = jnp.dot(m_ref[...], t_ref[...], preferred_element_type=jnp.float32)
    den = prod.sum(axis=-1, keepdims=True)
    o_ref[...] = prod / den

  return pl.pallas_call(
      body,
      out_shape=jax.ShapeDtypeStruct((_N_PRED, _N_CLS), jnp.float32),
  )(m, targets)


def kernel(predseg, targetseg, targets):
  m = _sc_hist_iou(predseg.astype(jnp.int32), targetseg.astype(jnp.int32))
  return _tc_finish(m, targets.astype(jnp.float32))


# speedup vs baseline: 1.0859x; 1.0859x over previous
import functools

import jax
import jax.numpy as jnp
from jax import lax
from jax.experimental import pallas as pl
from jax.experimental.pallas import tpu as pltpu
from jax.experimental.pallas import tpu_sc as plsc

_N_PRED = 512
_N_CLS = 128
_B = 16
_S = 4096
_NT = _B * _N_CLS
_NW = 32
_PPW = _N_PRED // _NW
_L = 16
_KW = _PPW * _NT
_HW = _N_PRED * _NT
_ZW = 4096
_NCH = _S // _N_CLS


def _sc_hist_iou(pred, targ):
  mesh = plsc.VectorSubcoreMesh(core_axis_name="c", subcore_axis_name="s")

  @functools.partial(
      pl.kernel,
      out_type=jax.ShapeDtypeStruct((_N_PRED, _N_CLS), jnp.float32),
      mesh=mesh,
      compiler_params=pltpu.CompilerParams(
          use_tc_tiling_on_sc=False, needs_layout_passes=False),
      scratch_types=[
          pltpu.VMEM((_S,), jnp.int32),
          pltpu.VMEM((_S,), jnp.int32),
          pltpu.VMEM((_NCH, _N_CLS), jnp.int32),
          pltpu.VMEM((_KW,), jnp.float32),
          pltpu.VMEM((_NT,), jnp.float32),
          pltpu.VMEM((_N_CLS,), jnp.float32),
          pltpu.VMEM((_PPW, _N_CLS), jnp.float32),
          pltpu.VMEM((_ZW,), jnp.float32),
          pltpu.VMEM((_N_CLS,), jnp.float32),
          pltpu.VMEM_SHARED((_HW,), jnp.float32),
          pltpu.VMEM_SHARED((_NT,), jnp.float32),
          pltpu.SemaphoreType.DMA,
          pltpu.SemaphoreType.DMA,
          pltpu.SemaphoreType.DMA,
      ],
  )
  def run(pred_hbm, targ_hbm, out_hbm, pbuf, tbuf, kbuf, hblk, ntv, ntloc,
          outv, zbuf, oneb, hsh, ntsh, semk, semz, sems):
    cid = lax.axis_index("c")
    sid = lax.axis_index("s")
    wid = sid * 2 + cid
    lo = wid * _PPW
    zer = jnp.zeros((_L,), jnp.float32)
    one = jnp.ones((_L,), jnp.float32)

    cpp = pltpu.async_copy(pred_hbm.at[sid], pbuf, semk)
    cpt = pltpu.async_copy(targ_hbm.at[sid], tbuf, semk)

    def zero_z(i, c):
      for k in range(8):
        zbuf[pl.ds(i * (8 * _L) + k * _L, _L)] = zer
      return c
    lax.fori_loop(0, _ZW // (8 * _L), zero_z, 0)
    for k in range(_N_CLS // _L):
      ntloc[pl.ds(k * _L, _L)] = zer
      oneb[pl.ds(k * _L, _L)] = one

    zcopies = [
        pltpu.async_copy(
            zbuf, hsh.at[pl.ds(sid * (_HW // 16) + i * _ZW, _ZW)], semz)
        for i in range(_HW // 16 // _ZW)
    ]
    cpp.wait()
    cpt.wait()

    boff = sid * _N_CLS

    def key_build(j, c):
      for k in range(_N_CLS // _L):
        o = j * _N_CLS + k * _L
        pv = pbuf[pl.ds(o, _L)]
        tv = tbuf[pl.ds(o, _L)]
        kbuf[j, pl.ds(k * _L, _L)] = (pv << 11) + (tv + boff)
        plsc.addupdate_scatter(ntloc, [tv], one)
      return c
    lax.fori_loop(0, _NCH, key_build, 0)
    pltpu.sync_copy(ntloc, ntsh.at[pl.ds(sid * _N_CLS, _N_CLS)])

    for cp in zcopies:
      cp.wait()
    plsc.subcore_barrier()

    scopies = [
        pltpu.async_copy(oneb, hsh.at[kbuf.at[j]], sems, add=True)
        for j in range(_NCH)
    ]
    for cp in scopies:
      cp.wait()
    plsc.subcore_barrier()

    pltpu.sync_copy(hsh.at[pl.ds(wid * _KW, _KW)], hblk)
    pltpu.sync_copy(ntsh, ntv)

    def row(r, c):
      rbase = r * _NT

      def np_sum(v, acc):
        s = acc
        for k in range(8):
          s = s + hblk[pl.ds(rbase + v * (8 * _L) + k * _L, _L)]
        return s
      np_lanes = lax.fori_loop(0, _NT // (8 * _L), np_sum, zer)
      np_v = jnp.full((_L,), jnp.sum(np_lanes))

      def acc_b(b, accs):
        hb = rbase + b * _N_CLS
        nb = b * _N_CLS
        out = []
        for j in range(_N_CLS // _L):
          h = hblk[pl.ds(hb + j * _L, _L)]
          nt = ntv[pl.ds(nb + j * _L, _L)]
          iou = jnp.where(h > 0.0, h / ((np_v + nt) - h), 0.0)
          out.append(accs[j] + iou)
        return tuple(out)

      accs = lax.fori_loop(0, _B, acc_b, tuple(zer for _ in range(_N_CLS // _L)))
      for j in range(_N_CLS // _L):
        outv[r, pl.ds(j * _L, _L)] = accs[j]
      return c
    lax.fori_loop(0, _PPW, row, 0)

    pltpu.sync_copy(outv, out_hbm.at[pl.ds(lo, _PPW)])

  return run(pred, targ)


def _tc_finish(m, targets):
  def body(m_ref, t_ref, o_ref):
    prod = jnp.dot(m_ref[...], t_ref[...], preferred_element_type=jnp.float32)
    den = prod.sum(axis=-1, keepdims=True)
    o_ref[...] = prod / den

  return pl.pallas_call(
      body,
      out_shape=jax.ShapeDtypeStruct((_N_PRED, _N_CLS), jnp.float32),
  )(m, targets)


def kernel(predseg, targetseg, targets):
  m = _sc_hist_iou(predseg.astype(jnp.int32), targetseg.astype(jnp.int32))
  return _tc_finish(m, targets.astype(jnp.float32))
